# baseline (speedup 1.0000x reference)
import jax
import jax.numpy as jnp
from jax import lax
from jax.experimental import pallas as pl
from jax.experimental.pallas import tpu as pltpu
from jax.experimental.pallas import tpu_sc as plsc

N_NODES = 10000
N_EDGES = 320000
D = 128

NC = 2
NS = 16
NW = NC * NS
EPW = N_EDGES // NW
CH = 128
NCHP = 80
PH = 2
CPP = NCHP // PH
N_PAD = 10240
RPT = N_PAD // NS
ZCH = 128
NZ = RPT // ZCH

import functools


@functools.cache
def _sc_mesh():
    return plsc.VectorSubcoreMesh(core_axis_name="c", subcore_axis_name="s",
                                  num_cores=NC, num_subcores=NS)


def _deg_body(dst_hbm, zeros_hbm, ones_hbm, out_hbm, dst_v, buf_v, acc_sh):
    c = lax.axis_index("c")
    s = lax.axis_index("s")
    base = s * RPT
    pltpu.sync_copy(zeros_hbm, buf_v)

    def initstep(k, carry):
        pltpu.sync_copy(buf_v, acc_sh.at[pl.ds(base + k * ZCH, ZCH)])
        return carry

    lax.fori_loop(0, NZ, initstep, 0)
    w = c * NS + s
    pltpu.sync_copy(dst_hbm.at[w], dst_v)
    pltpu.sync_copy(ones_hbm, buf_v)
    plsc.subcore_barrier()

    def step(j, carry):
        pltpu.sync_copy(buf_v, acc_sh.at[dst_v.at[j]], add=True)
        return carry

    lax.fori_loop(0, NCHP, step, 0)
    plsc.subcore_barrier()

    def drainstep(k, carry):
        pltpu.sync_copy(acc_sh.at[pl.ds(base + k * ZCH, ZCH)], buf_v)
        pltpu.sync_copy(buf_v, out_hbm.at[c, pl.ds(base + k * ZCH, ZCH)])
        return carry

    lax.fori_loop(0, NZ, drainstep, 0)


@functools.cache
def _deg_kernel():
    return pl.kernel(
        _deg_body,
        out_type=jax.ShapeDtypeStruct((NC, N_PAD, D), jnp.float32),
        mesh=_sc_mesh(),
        scratch_types=[
            pltpu.VMEM((NCHP, CH), jnp.int32),
            pltpu.VMEM((ZCH, D), jnp.float32),
            pltpu.VMEM_SHARED((N_PAD, D), jnp.float32),
        ],
    )


def _agg_body(g_hbm, src_hbm, dst_hbm, zeros_hbm, out_hbm,
              src_v, dst_v, rows_v, acc_sh, sem):
    c = lax.axis_index("c")
    s = lax.axis_index("s")
    base = s * RPT
    pltpu.sync_copy(zeros_hbm, rows_v)

    def initstep(k, carry):
        pltpu.sync_copy(rows_v, acc_sh.at[pl.ds(base + k * ZCH, ZCH)])
        return carry

    lax.fori_loop(0, NZ, initstep, 0)
    w = c * NS + s
    plsc.subcore_barrier()
    for p in range(PH):
        pltpu.sync_copy(src_hbm.at[w, pl.ds(p * CPP, CPP)], src_v)
        pltpu.sync_copy(dst_hbm.at[w, pl.ds(p * CPP, CPP)], dst_v)

        def step(j, carry):
            pltpu.async_copy(g_hbm.at[src_v.at[j]], rows_v, sem).wait()
            pltpu.sync_copy(rows_v, acc_sh.at[dst_v.at[j]], add=True)
            return carry

        lax.fori_loop(0, CPP, step, 0)
    plsc.subcore_barrier()

    def drainstep(k, carry):
        pltpu.sync_copy(acc_sh.at[pl.ds(base + k * ZCH, ZCH)], rows_v)
        pltpu.sync_copy(rows_v, out_hbm.at[c, pl.ds(base + k * ZCH, ZCH)])
        return carry

    lax.fori_loop(0, NZ, drainstep, 0)


@functools.cache
def _agg_kernel():
    return pl.kernel(
        _agg_body,
        out_type=jax.ShapeDtypeStruct((NC, N_PAD, D), jnp.float32),
        mesh=_sc_mesh(),
        scratch_types=[
            pltpu.VMEM((CPP, CH), jnp.int32),
            pltpu.VMEM((CPP, CH), jnp.int32),
            pltpu.VMEM((CH, D), jnp.float32),
            pltpu.VMEM_SHARED((N_PAD, D), jnp.float32),
            pltpu.SemaphoreType.DMA,
        ],
    )


BLK = 1000
GRID = N_NODES // BLK


def _dinv_of(da_ref, db_ref):
    deg = 1.0 + da_ref[0][:, 0:1] + db_ref[0][:, 0:1]
    return lax.rsqrt(deg)


def _tc1_body(x_ref, w_ref, da_ref, db_ref, g_ref):
    dinv = _dinv_of(da_ref, db_ref)
    h = jnp.dot(x_ref[...], w_ref[...], preferred_element_type=jnp.float32)
    g_ref[...] = h * dinv


def _tc2_body(a0_ref, a1_ref, g1_ref, da_ref, db_ref, b_ref, w_ref, g2_ref):
    dinv = _dinv_of(da_ref, db_ref)
    t = (a0_ref[0] + a1_ref[0] + g1_ref[...]) * dinv + b_ref[...]
    t = jnp.maximum(t, 0.0)
    h = jnp.dot(t, w_ref[...], preferred_element_type=jnp.float32)
    g2_ref[...] = h * dinv


def _tc3_body(a0_ref, a1_ref, g2_ref, da_ref, db_ref, b_ref, out_ref):
    dinv = _dinv_of(da_ref, db_ref)
    out_ref[...] = (a0_ref[0] + a1_ref[0] + g2_ref[...]) * dinv + b_ref[...]


def _rows(i):
    return (i, 0)


def _plane0(i):
    return (0, i, 0)


def _plane1(i):
    return (1, i, 0)


def _whole(i):
    return (0, 0)


_rows_spec = pl.BlockSpec((BLK, D), _rows)
_dega_spec = pl.BlockSpec((1, BLK, D), _plane0)
_degb_spec = pl.BlockSpec((1, BLK, D), _plane1)
_agg0_spec = pl.BlockSpec((1, BLK, D), _plane0)
_agg1_spec = pl.BlockSpec((1, BLK, D), _plane1)
_mat_spec = pl.BlockSpec((D, D), _whole)
_bias_spec = pl.BlockSpec((1, D), _whole)
_out_sds = jax.ShapeDtypeStruct((N_NODES, D), jnp.float32)

_tc1 = pl.pallas_call(
    _tc1_body, grid=(GRID,),
    in_specs=[_rows_spec, _mat_spec, _dega_spec, _degb_spec],
    out_specs=_rows_spec, out_shape=_out_sds)

_tc2 = pl.pallas_call(
    _tc2_body, grid=(GRID,),
    in_specs=[_agg0_spec, _agg1_spec, _rows_spec, _dega_spec, _degb_spec,
              _bias_spec, _mat_spec],
    out_specs=_rows_spec, out_shape=_out_sds)

_tc3 = pl.pallas_call(
    _tc3_body, grid=(GRID,),
    in_specs=[_agg0_spec, _agg1_spec, _rows_spec, _dega_spec, _degb_spec,
              _bias_spec],
    out_specs=_rows_spec, out_shape=_out_sds)


def kernel(x, edge_index, W1, b1, W2, b2):
    ei = edge_index.astype(jnp.int32)
    npad = NCHP * CH - EPW
    src2 = ei[0].reshape(NW, EPW)
    dst2 = ei[1].reshape(NW, EPW)
    pad_s = jnp.zeros((NW, npad), jnp.int32)
    pad_d = jnp.full((NW, npad), N_PAD - 1, jnp.int32)
    src3 = jnp.concatenate([src2, pad_s], axis=1).reshape(NW, NCHP, CH)
    dst3 = jnp.concatenate([dst2, pad_d], axis=1).reshape(NW, NCHP, CH)
    zeros_d = jnp.zeros((ZCH, D), jnp.float32)
    ones_d = jnp.ones((ZCH, D), jnp.float32)

    degp = _deg_kernel()(dst3, zeros_d, ones_d)
    g1 = _tc1(x, W1, degp, degp)
    agg1 = _agg_kernel()(g1, src3, dst3, zeros_d)
    g2 = _tc2(agg1, agg1, g1, degp, degp, b1.reshape(1, D), W2)
    agg2 = _agg_kernel()(g2, src3, dst3, zeros_d)
    out = _tc3(agg2, agg2, g2, degp, degp, b2.reshape(1, D))
    return out


# speedup vs baseline: 1.0069x; 1.0069x over previous
import jax
import jax.numpy as jnp
from jax import lax
from jax.experimental import pallas as pl
from jax.experimental.pallas import tpu as pltpu
from jax.experimental.pallas import tpu_sc as plsc

N_NODES = 10000
N_EDGES = 320000
D = 128

NC = 2
NS = 16
NW = NC * NS
EPW = N_EDGES // NW
CH = 80
NCHP = 128
PH = 2
CPP = NCHP // PH
N_PAD = 10240
RPT = N_PAD // NS
ZCH = 80
NZ = RPT // ZCH

import functools


@functools.cache
def _sc_mesh():
    return plsc.VectorSubcoreMesh(core_axis_name="c", subcore_axis_name="s",
                                  num_cores=NC, num_subcores=NS)


def _deg_body(dst_hbm, zeros_hbm, ones_hbm, out_hbm, dst_v, buf_v, acc_sh):
    c = lax.axis_index("c")
    s = lax.axis_index("s")
    base = s * RPT
    pltpu.sync_copy(zeros_hbm, buf_v)

    def initstep(k, carry):
        pltpu.sync_copy(buf_v, acc_sh.at[pl.ds(base + k * ZCH, ZCH)])
        return carry

    lax.fori_loop(0, NZ, initstep, 0)
    w = c * NS + s
    pltpu.sync_copy(dst_hbm.at[w], dst_v)
    pltpu.sync_copy(ones_hbm, buf_v)
    plsc.subcore_barrier()

    def step(j, carry):
        pltpu.sync_copy(buf_v, acc_sh.at[dst_v.at[j]], add=True)
        return carry

    lax.fori_loop(0, NCHP, step, 0)
    plsc.subcore_barrier()

    def drainstep(k, carry):
        pltpu.sync_copy(acc_sh.at[pl.ds(base + k * ZCH, ZCH)], buf_v)
        pltpu.sync_copy(buf_v, out_hbm.at[c, pl.ds(base + k * ZCH, ZCH)])
        return carry

    lax.fori_loop(0, NZ, drainstep, 0)


@functools.cache
def _deg_kernel():
    return pl.kernel(
        _deg_body,
        out_type=jax.ShapeDtypeStruct((NC, N_PAD, D), jnp.float32),
        mesh=_sc_mesh(),
        scratch_types=[
            pltpu.VMEM((NCHP, CH), jnp.int32),
            pltpu.VMEM((ZCH, D), jnp.float32),
            pltpu.VMEM_SHARED((N_PAD, D), jnp.float32),
        ],
    )


def _agg_body(g_hbm, src_hbm, dst_hbm, zeros_hbm, out_hbm,
              src_v, dst_v, rows0_v, rows1_v, acc_sh, sem):
    c = lax.axis_index("c")
    s = lax.axis_index("s")
    base = s * RPT
    pltpu.sync_copy(zeros_hbm, rows0_v)

    def initstep(k, carry):
        pltpu.sync_copy(rows0_v, acc_sh.at[pl.ds(base + k * ZCH, ZCH)])
        return carry

    lax.fori_loop(0, NZ, initstep, 0)
    w = c * NS + s
    plsc.subcore_barrier()
    for p in range(PH):
        pltpu.sync_copy(src_hbm.at[w, pl.ds(p * CPP, CPP)], src_v)
        pltpu.sync_copy(dst_hbm.at[w, pl.ds(p * CPP, CPP)], dst_v)

        def step(i, carry):
            j = 2 * i
            cp0 = pltpu.async_copy(g_hbm.at[src_v.at[j]], rows0_v, sem)
            cp1 = pltpu.async_copy(g_hbm.at[src_v.at[j + 1]], rows1_v, sem)
            cp0.wait()
            cp1.wait()
            pltpu.sync_copy(rows0_v, acc_sh.at[dst_v.at[j]], add=True)
            pltpu.sync_copy(rows1_v, acc_sh.at[dst_v.at[j + 1]], add=True)
            return carry

        lax.fori_loop(0, CPP // 2, step, 0)
    plsc.subcore_barrier()

    def drainstep(k, carry):
        pltpu.sync_copy(acc_sh.at[pl.ds(base + k * ZCH, ZCH)], rows0_v)
        pltpu.sync_copy(rows0_v, out_hbm.at[c, pl.ds(base + k * ZCH, ZCH)])
        return carry

    lax.fori_loop(0, NZ, drainstep, 0)


@functools.cache
def _agg_kernel():
    return pl.kernel(
        _agg_body,
        out_type=jax.ShapeDtypeStruct((NC, N_PAD, D), jnp.float32),
        mesh=_sc_mesh(),
        scratch_types=[
            pltpu.VMEM((CPP, CH), jnp.int32),
            pltpu.VMEM((CPP, CH), jnp.int32),
            pltpu.VMEM((CH, D), jnp.float32),
            pltpu.VMEM((CH, D), jnp.float32),
            pltpu.VMEM_SHARED((N_PAD, D), jnp.float32),
            pltpu.SemaphoreType.DMA,
        ],
    )


BLK = 1000
GRID = N_NODES // BLK


def _dinv_of(da_ref, db_ref):
    deg = 1.0 + da_ref[0][:, 0:1] + db_ref[0][:, 0:1]
    return lax.rsqrt(deg)


def _tc1_body(x_ref, w_ref, da_ref, db_ref, g_ref):
    dinv = _dinv_of(da_ref, db_ref)
    h = jnp.dot(x_ref[...], w_ref[...], preferred_element_type=jnp.float32)
    g_ref[...] = h * dinv


def _tc2_body(a0_ref, a1_ref, g1_ref, da_ref, db_ref, b_ref, w_ref, g2_ref):
    dinv = _dinv_of(da_ref, db_ref)
    t = (a0_ref[0] + a1_ref[0] + g1_ref[...]) * dinv + b_ref[...]
    t = jnp.maximum(t, 0.0)
    h = jnp.dot(t, w_ref[...], preferred_element_type=jnp.float32)
    g2_ref[...] = h * dinv


def _tc3_body(a0_ref, a1_ref, g2_ref, da_ref, db_ref, b_ref, out_ref):
    dinv = _dinv_of(da_ref, db_ref)
    out_ref[...] = (a0_ref[0] + a1_ref[0] + g2_ref[...]) * dinv + b_ref[...]


def _rows(i):
    return (i, 0)


def _plane0(i):
    return (0, i, 0)


def _plane1(i):
    return (1, i, 0)


def _whole(i):
    return (0, 0)


_rows_spec = pl.BlockSpec((BLK, D), _rows)
_dega_spec = pl.BlockSpec((1, BLK, D), _plane0)
_degb_spec = pl.BlockSpec((1, BLK, D), _plane1)
_agg0_spec = pl.BlockSpec((1, BLK, D), _plane0)
_agg1_spec = pl.BlockSpec((1, BLK, D), _plane1)
_mat_spec = pl.BlockSpec((D, D), _whole)
_bias_spec = pl.BlockSpec((1, D), _whole)
_out_sds = jax.ShapeDtypeStruct((N_NODES, D), jnp.float32)

_tc1 = pl.pallas_call(
    _tc1_body, grid=(GRID,),
    in_specs=[_rows_spec, _mat_spec, _dega_spec, _degb_spec],
    out_specs=_rows_spec, out_shape=_out_sds)

_tc2 = pl.pallas_call(
    _tc2_body, grid=(GRID,),
    in_specs=[_agg0_spec, _agg1_spec, _rows_spec, _dega_spec, _degb_spec,
              _bias_spec, _mat_spec],
    out_specs=_rows_spec, out_shape=_out_sds)

_tc3 = pl.pallas_call(
    _tc3_body, grid=(GRID,),
    in_specs=[_agg0_spec, _agg1_spec, _rows_spec, _dega_spec, _degb_spec,
              _bias_spec],
    out_specs=_rows_spec, out_shape=_out_sds)


def kernel(x, edge_index, W1, b1, W2, b2):
    ei = edge_index.astype(jnp.int32)
    npad = NCHP * CH - EPW
    src2 = ei[0].reshape(NW, EPW)
    dst2 = ei[1].reshape(NW, EPW)
    pad_s = jnp.zeros((NW, npad), jnp.int32)
    pad_d = jnp.full((NW, npad), N_PAD - 1, jnp.int32)
    src3 = jnp.concatenate([src2, pad_s], axis=1).reshape(NW, NCHP, CH)
    dst3 = jnp.concatenate([dst2, pad_d], axis=1).reshape(NW, NCHP, CH)
    zeros_d = jnp.zeros((ZCH, D), jnp.float32)
    ones_d = jnp.ones((ZCH, D), jnp.float32)

    degp = _deg_kernel()(dst3, zeros_d, ones_d)
    g1 = _tc1(x, W1, degp, degp)
    agg1 = _agg_kernel()(g1, src3, dst3, zeros_d)
    g2 = _tc2(agg1, agg1, g1, degp, degp, b1.reshape(1, D), W2)
    agg2 = _agg_kernel()(g2, src3, dst3, zeros_d)
    out = _tc3(agg2, agg2, g2, degp, degp, b2.reshape(1, D))
    return out


# speedup vs baseline: 2.1620x; 2.1472x over previous
import jax
import jax.numpy as jnp
from jax import lax
from jax.experimental import pallas as pl
from jax.experimental.pallas import tpu as pltpu
from jax.experimental.pallas import tpu_sc as plsc

N_NODES = 10000
N_EDGES = 320000
D = 128

NC = 2
NS = 16
NW = NC * NS
EPW = N_EDGES // NW
CH = 80
NCHP = 128
PH = 2
CPP = NCHP // PH
N_PAD = 10240
RPT = N_PAD // NS
ZCH = 80
NZ = RPT // ZCH

import functools


@functools.cache
def _sc_mesh():
    return plsc.VectorSubcoreMesh(core_axis_name="c", subcore_axis_name="s",
                                  num_cores=NC, num_subcores=NS)


def _deg_body(dst_hbm, zeros_hbm, ones_hbm, out_hbm, dst_v, buf_v, acc_sh):
    c = lax.axis_index("c")
    s = lax.axis_index("s")
    base = s * RPT
    pltpu.sync_copy(zeros_hbm, buf_v)

    def initstep(k, carry):
        pltpu.sync_copy(buf_v, acc_sh.at[pl.ds(base + k * ZCH, ZCH)])
        return carry

    lax.fori_loop(0, NZ, initstep, 0)
    w = c * NS + s
    pltpu.sync_copy(dst_hbm.at[w], dst_v)
    pltpu.sync_copy(ones_hbm, buf_v)
    plsc.subcore_barrier()

    def step(j, carry):
        pltpu.sync_copy(buf_v, acc_sh.at[dst_v.at[j]], add=True)
        return carry

    lax.fori_loop(0, NCHP, step, 0)
    plsc.subcore_barrier()

    def drainstep(k, carry):
        pltpu.sync_copy(acc_sh.at[pl.ds(base + k * ZCH, ZCH)], buf_v)
        pltpu.sync_copy(buf_v, out_hbm.at[c, pl.ds(base + k * ZCH, ZCH)])
        return carry

    lax.fori_loop(0, NZ, drainstep, 0)


@functools.cache
def _deg_kernel():
    return pl.kernel(
        _deg_body,
        out_type=jax.ShapeDtypeStruct((NC, N_PAD, D), jnp.float32),
        mesh=_sc_mesh(),
        scratch_types=[
            pltpu.VMEM((NCHP, CH), jnp.int32),
            pltpu.VMEM((ZCH, D), jnp.float32),
            pltpu.VMEM_SHARED((N_PAD, D), jnp.float32),
        ],
    )


def _agg_body(g_hbm, src_hbm, dst_hbm, zeros_hbm, out_hbm,
              src_v, dst_v, rows0_v, rows1_v, acc_sh, sem):
    c = lax.axis_index("c")
    s = lax.axis_index("s")
    base = s * RPT
    pltpu.sync_copy(zeros_hbm, rows0_v)

    def initstep(k, carry):
        pltpu.sync_copy(rows0_v, acc_sh.at[pl.ds(base + k * ZCH, ZCH)])
        return carry

    lax.fori_loop(0, NZ, initstep, 0)
    w = c * NS + s
    plsc.subcore_barrier()
    for p in range(PH):
        pltpu.sync_copy(src_hbm.at[w, pl.ds(p * CPP, CPP)], src_v)
        pltpu.sync_copy(dst_hbm.at[w, pl.ds(p * CPP, CPP)], dst_v)

        def step(i, carry):
            j = 2 * i
            cp0 = pltpu.async_copy(g_hbm.at[src_v.at[j]], rows0_v, sem)
            cp1 = pltpu.async_copy(g_hbm.at[src_v.at[j + 1]], rows1_v, sem)
            cp0.wait()
            cp1.wait()
            pltpu.sync_copy(rows0_v, acc_sh.at[dst_v.at[j]], add=True)
            pltpu.sync_copy(rows1_v, acc_sh.at[dst_v.at[j + 1]], add=True)
            return carry

        lax.fori_loop(0, CPP // 2, step, 0)
    plsc.subcore_barrier()

    def drainstep(k, carry):
        pltpu.sync_copy(acc_sh.at[pl.ds(base + k * ZCH, ZCH)], rows0_v)
        pltpu.sync_copy(rows0_v, out_hbm.at[c, pl.ds(base + k * ZCH, ZCH)])
        return carry

    lax.fori_loop(0, NZ, drainstep, 0)


@functools.cache
def _agg_kernel():
    return pl.kernel(
        _agg_body,
        out_type=jax.ShapeDtypeStruct((NC, N_PAD, D), jnp.float32),
        mesh=_sc_mesh(),
        scratch_types=[
            pltpu.VMEM((CPP, CH), jnp.int32),
            pltpu.VMEM((CPP, CH), jnp.int32),
            pltpu.VMEM((CH, D), jnp.float32),
            pltpu.VMEM((CH, D), jnp.float32),
            pltpu.VMEM_SHARED((N_PAD, D), jnp.float32),
            pltpu.SemaphoreType.DMA,
        ],
    )


BLK = 1000
GRID = N_NODES // BLK


def _dinv_of(da_ref, db_ref):
    deg = 1.0 + da_ref[0][:, 0:1] + db_ref[0][:, 0:1]
    return lax.rsqrt(deg)


def _tc1_body(x_ref, w_ref, da_ref, db_ref, g_ref):
    dinv = _dinv_of(da_ref, db_ref)
    h = jnp.dot(x_ref[...], w_ref[...], preferred_element_type=jnp.float32)
    g_ref[...] = h * dinv


def _tc2_body(a0_ref, a1_ref, g1_ref, da_ref, db_ref, b_ref, w_ref, g2_ref):
    dinv = _dinv_of(da_ref, db_ref)
    t = (a0_ref[0] + a1_ref[0] + g1_ref[...]) * dinv + b_ref[...]
    t = jnp.maximum(t, 0.0)
    h = jnp.dot(t, w_ref[...], preferred_element_type=jnp.float32)
    g2_ref[...] = h * dinv


def _tc3_body(a0_ref, a1_ref, g2_ref, da_ref, db_ref, b_ref, out_ref):
    dinv = _dinv_of(da_ref, db_ref)
    out_ref[...] = (a0_ref[0] + a1_ref[0] + g2_ref[...]) * dinv + b_ref[...]


def _rows(i):
    return (i, 0)


def _plane0(i):
    return (0, i, 0)


def _plane1(i):
    return (1, i, 0)


def _whole(i):
    return (0, 0)


_rows_spec = pl.BlockSpec((BLK, D), _rows)
_dega_spec = pl.BlockSpec((1, BLK, D), _plane0)
_degb_spec = pl.BlockSpec((1, BLK, D), _plane1)
_agg0_spec = pl.BlockSpec((1, BLK, D), _plane0)
_agg1_spec = pl.BlockSpec((1, BLK, D), _plane1)
_mat_spec = pl.BlockSpec((D, D), _whole)
_bias_spec = pl.BlockSpec((1, D), _whole)
_out_sds = jax.ShapeDtypeStruct((N_NODES, D), jnp.float32)

_tc1 = pl.pallas_call(
    _tc1_body, grid=(GRID,),
    in_specs=[_rows_spec, _mat_spec, _dega_spec, _degb_spec],
    out_specs=_rows_spec, out_shape=_out_sds)

_tc2 = pl.pallas_call(
    _tc2_body, grid=(GRID,),
    in_specs=[_agg0_spec, _agg1_spec, _rows_spec, _dega_spec, _degb_spec,
              _bias_spec, _mat_spec],
    out_specs=_rows_spec, out_shape=_out_sds)

_tc3 = pl.pallas_call(
    _tc3_body, grid=(GRID,),
    in_specs=[_agg0_spec, _agg1_spec, _rows_spec, _dega_spec, _degb_spec,
              _bias_spec],
    out_specs=_rows_spec, out_shape=_out_sds)


def kernel(x, edge_index, W1, b1, W2, b2):
    ei = edge_index.astype(jnp.int32)
    npad = NCHP * CH - EPW
    src2 = ei[0].reshape(NW, EPW)
    dst2 = ei[1].reshape(NW, EPW)
    pad_s = jnp.broadcast_to(jnp.arange(npad, dtype=jnp.int32), (NW, npad))
    pad_d = jnp.broadcast_to(
        N_NODES + jnp.arange(npad, dtype=jnp.int32), (NW, npad))
    src3 = jnp.concatenate([src2, pad_s], axis=1).reshape(NW, NCHP, CH)
    dst3 = jnp.concatenate([dst2, pad_d], axis=1).reshape(NW, NCHP, CH)
    zeros_d = jnp.zeros((ZCH, D), jnp.float32)
    ones_d = jnp.ones((ZCH, D), jnp.float32)

    degp = _deg_kernel()(dst3, zeros_d, ones_d)
    g1 = _tc1(x, W1, degp, degp)
    agg1 = _agg_kernel()(g1, src3, dst3, zeros_d)
    g2 = _tc2(agg1, agg1, g1, degp, degp, b1.reshape(1, D), W2)
    agg2 = _agg_kernel()(g2, src3, dst3, zeros_d)
    out = _tc3(agg2, agg2, g2, degp, degp, b2.reshape(1, D))
    return out


# speedup vs baseline: 2.6869x; 1.2428x over previous
import jax
import jax.numpy as jnp
from jax import lax
from jax.experimental import pallas as pl
from jax.experimental.pallas import tpu as pltpu
from jax.experimental.pallas import tpu_sc as plsc

N_NODES = 10000
N_EDGES = 320000
D = 128

NC = 2
NS = 16
NW = NC * NS
EPW = N_EDGES // NW
CH = 80
NCHP = 128
PH = 2
CPP = NCHP // PH
N_PAD = 10240
RPT = N_PAD // NS
ZCH = 80
NZ = RPT // ZCH

import functools


@functools.cache
def _sc_mesh():
    return plsc.VectorSubcoreMesh(core_axis_name="c", subcore_axis_name="s",
                                  num_cores=NC, num_subcores=NS)


def _deg_body(dst_hbm, zeros_hbm, ones_hbm, out_hbm, dst_v, buf_v, acc_sh):
    c = lax.axis_index("c")
    s = lax.axis_index("s")
    base = s * RPT
    pltpu.sync_copy(zeros_hbm, buf_v)

    def initstep(k, carry):
        pltpu.sync_copy(buf_v, acc_sh.at[pl.ds(base + k * ZCH, ZCH)])
        return carry

    lax.fori_loop(0, NZ, initstep, 0)
    w = c * NS + s
    pltpu.sync_copy(dst_hbm.at[w], dst_v)
    pltpu.sync_copy(ones_hbm, buf_v)
    plsc.subcore_barrier()

    def step(j, carry):
        pltpu.sync_copy(buf_v, acc_sh.at[dst_v.at[j]], add=True)
        return carry

    lax.fori_loop(0, NCHP, step, 0)
    plsc.subcore_barrier()

    def drainstep(k, carry):
        pltpu.sync_copy(acc_sh.at[pl.ds(base + k * ZCH, ZCH)], buf_v)
        pltpu.sync_copy(buf_v, out_hbm.at[c, pl.ds(base + k * ZCH, ZCH)])
        return carry

    lax.fori_loop(0, NZ, drainstep, 0)


@functools.cache
def _deg_kernel():
    return pl.kernel(
        _deg_body,
        out_type=jax.ShapeDtypeStruct((NC, N_PAD, D), jnp.float32),
        mesh=_sc_mesh(),
        scratch_types=[
            pltpu.VMEM((NCHP, CH), jnp.int32),
            pltpu.VMEM((ZCH, D), jnp.float32),
            pltpu.VMEM_SHARED((N_PAD, D), jnp.float32),
        ],
    )


def _agg_body(g_hbm, src_hbm, dst_hbm, zeros_hbm, out_hbm,
              src_v, dst_v, rows0_v, rows1_v, acc_sh, sem, sem1):
    c = lax.axis_index("c")
    s = lax.axis_index("s")
    base = s * RPT
    pltpu.sync_copy(zeros_hbm, rows0_v)

    def initstep(k, carry):
        pltpu.sync_copy(rows0_v, acc_sh.at[pl.ds(base + k * ZCH, ZCH)])
        return carry

    lax.fori_loop(0, NZ, initstep, 0)
    w = c * NS + s
    plsc.subcore_barrier()
    for p in range(PH):
        pltpu.sync_copy(src_hbm.at[w, pl.ds(p * CPP, CPP)], src_v)
        pltpu.sync_copy(dst_hbm.at[w, pl.ds(p * CPP, CPP)], dst_v)

        pltpu.async_copy(g_hbm.at[src_v.at[0]], rows0_v, sem)

        def step(i, carry):
            j = 2 * i
            pltpu.async_copy(g_hbm.at[src_v.at[j + 1]], rows1_v, sem1)
            pltpu.make_async_copy(g_hbm.at[src_v.at[j]], rows0_v, sem).wait()
            pltpu.sync_copy(rows0_v, acc_sh.at[dst_v.at[j]], add=True)
            pltpu.async_copy(g_hbm.at[src_v.at[j + 2]], rows0_v, sem)
            pltpu.make_async_copy(g_hbm.at[src_v.at[j + 1]], rows1_v,
                                  sem1).wait()
            pltpu.sync_copy(rows1_v, acc_sh.at[dst_v.at[j + 1]], add=True)
            return carry

        lax.fori_loop(0, CPP // 2 - 1, step, 0)
        pltpu.async_copy(g_hbm.at[src_v.at[CPP - 1]], rows1_v, sem1)
        pltpu.make_async_copy(g_hbm.at[src_v.at[CPP - 2]], rows0_v,
                              sem).wait()
        pltpu.sync_copy(rows0_v, acc_sh.at[dst_v.at[CPP - 2]], add=True)
        pltpu.make_async_copy(g_hbm.at[src_v.at[CPP - 1]], rows1_v,
                              sem1).wait()
        pltpu.sync_copy(rows1_v, acc_sh.at[dst_v.at[CPP - 1]], add=True)
    plsc.subcore_barrier()

    def drainstep(k, carry):
        pltpu.sync_copy(acc_sh.at[pl.ds(base + k * ZCH, ZCH)], rows0_v)
        pltpu.sync_copy(rows0_v, out_hbm.at[c, pl.ds(base + k * ZCH, ZCH)])
        return carry

    lax.fori_loop(0, NZ, drainstep, 0)


@functools.cache
def _agg_kernel():
    return pl.kernel(
        _agg_body,
        out_type=jax.ShapeDtypeStruct((NC, N_PAD, D), jnp.float32),
        mesh=_sc_mesh(),
        scratch_types=[
            pltpu.VMEM((CPP, CH), jnp.int32),
            pltpu.VMEM((CPP, CH), jnp.int32),
            pltpu.VMEM((CH, D), jnp.float32),
            pltpu.VMEM((CH, D), jnp.float32),
            pltpu.VMEM_SHARED((N_PAD, D), jnp.float32),
            pltpu.SemaphoreType.DMA,
            pltpu.SemaphoreType.DMA,
        ],
    )


BLK = 1000
GRID = N_NODES // BLK


def _dinv_of(da_ref, db_ref):
    deg = 1.0 + da_ref[0][:, 0:1] + db_ref[0][:, 0:1]
    return lax.rsqrt(deg)


def _tc1_body(x_ref, w_ref, da_ref, db_ref, g_ref):
    dinv = _dinv_of(da_ref, db_ref)
    h = jnp.dot(x_ref[...], w_ref[...], preferred_element_type=jnp.float32)
    g_ref[...] = h * dinv


def _tc2_body(a0_ref, a1_ref, g1_ref, da_ref, db_ref, b_ref, w_ref, g2_ref):
    dinv = _dinv_of(da_ref, db_ref)
    t = (a0_ref[0] + a1_ref[0] + g1_ref[...]) * dinv + b_ref[...]
    t = jnp.maximum(t, 0.0)
    h = jnp.dot(t, w_ref[...], preferred_element_type=jnp.float32)
    g2_ref[...] = h * dinv


def _tc3_body(a0_ref, a1_ref, g2_ref, da_ref, db_ref, b_ref, out_ref):
    dinv = _dinv_of(da_ref, db_ref)
    out_ref[...] = (a0_ref[0] + a1_ref[0] + g2_ref[...]) * dinv + b_ref[...]


def _rows(i):
    return (i, 0)


def _plane0(i):
    return (0, i, 0)


def _plane1(i):
    return (1, i, 0)


def _whole(i):
    return (0, 0)


_rows_spec = pl.BlockSpec((BLK, D), _rows)
_dega_spec = pl.BlockSpec((1, BLK, D), _plane0)
_degb_spec = pl.BlockSpec((1, BLK, D), _plane1)
_agg0_spec = pl.BlockSpec((1, BLK, D), _plane0)
_agg1_spec = pl.BlockSpec((1, BLK, D), _plane1)
_mat_spec = pl.BlockSpec((D, D), _whole)
_bias_spec = pl.BlockSpec((1, D), _whole)
_out_sds = jax.ShapeDtypeStruct((N_NODES, D), jnp.float32)

_tc1 = pl.pallas_call(
    _tc1_body, grid=(GRID,),
    in_specs=[_rows_spec, _mat_spec, _dega_spec, _degb_spec],
    out_specs=_rows_spec, out_shape=_out_sds)

_tc2 = pl.pallas_call(
    _tc2_body, grid=(GRID,),
    in_specs=[_agg0_spec, _agg1_spec, _rows_spec, _dega_spec, _degb_spec,
              _bias_spec, _mat_spec],
    out_specs=_rows_spec, out_shape=_out_sds)

_tc3 = pl.pallas_call(
    _tc3_body, grid=(GRID,),
    in_specs=[_agg0_spec, _agg1_spec, _rows_spec, _dega_spec, _degb_spec,
              _bias_spec],
    out_specs=_rows_spec, out_shape=_out_sds)


def kernel(x, edge_index, W1, b1, W2, b2):
    ei = edge_index.astype(jnp.int32)
    npad = NCHP * CH - EPW
    src2 = ei[0].reshape(NW, EPW)
    dst2 = ei[1].reshape(NW, EPW)
    pad_s = jnp.broadcast_to(jnp.arange(npad, dtype=jnp.int32), (NW, npad))
    pad_d = jnp.broadcast_to(
        N_NODES + jnp.arange(npad, dtype=jnp.int32), (NW, npad))
    src3 = jnp.concatenate([src2, pad_s], axis=1).reshape(NW, NCHP, CH)
    dst3 = jnp.concatenate([dst2, pad_d], axis=1).reshape(NW, NCHP, CH)
    zeros_d = jnp.zeros((ZCH, D), jnp.float32)
    ones_d = jnp.ones((ZCH, D), jnp.float32)

    degp = _deg_kernel()(dst3, zeros_d, ones_d)
    g1 = _tc1(x, W1, degp, degp)
    agg1 = _agg_kernel()(g1, src3, dst3, zeros_d)
    g2 = _tc2(agg1, agg1, g1, degp, degp, b1.reshape(1, D), W2)
    agg2 = _agg_kernel()(g2, src3, dst3, zeros_d)
    out = _tc3(agg2, agg2, g2, degp, degp, b2.reshape(1, D))
    return out
